# baseline (speedup 1.0000x reference)
import functools

import jax
import jax.numpy as jnp
from jax import lax
from jax.experimental import pallas as pl
from jax.experimental.pallas import tpu as pltpu
from jax.experimental.pallas import tpu_sc as plsc

N = 10000
E = 320000
D = 128
G = 128
NC = 2
NS = 16
NW = NC * NS
C = 128
KCH = 80
EPAD = NW * KCH * C
R = 10240
RPT = R // NS
TRASH = 10000
BR = 2560
ZR = 64
F32 = jnp.float32


def _agg_body(table, srcp, dstp, zrows, acc_out,
              src_v, dst_v, rows0, rows1, acc_sh, sem0, sem1):
    rows = (rows0, rows1)
    sems = (sem0, sem1)
    c = lax.axis_index("c")
    s = lax.axis_index("s")
    wid = s * NC + c
    rbase = s * RPT
    for z in range(RPT // ZR):
        pltpu.sync_copy(zrows, acc_sh.at[pl.ds(rbase + z * ZR, ZR)])
    pltpu.sync_copy(srcp.at[wid], src_v)
    pltpu.sync_copy(dstp.at[wid], dst_v)
    plsc.subcore_barrier()

    def step(j, carry):
        pltpu.async_copy(table.at[src_v.at[j]], rows0, sem0).wait()
        pltpu.sync_copy(rows0, acc_sh.at[dst_v.at[j]], add=True)
        return carry

    lax.fori_loop(0, KCH, step, 0)
    plsc.subcore_barrier()
    pltpu.sync_copy(acc_sh.at[pl.ds(rbase, RPT)],
                    acc_out.at[c, pl.ds(rbase, RPT)])


_sc_aggregate = pl.kernel(
    _agg_body,
    out_type=jax.ShapeDtypeStruct((NC, R, D), F32),
    mesh=plsc.VectorSubcoreMesh(core_axis_name="c", subcore_axis_name="s",
                                num_cores=NC, num_subcores=NS),
    scratch_types=[
        pltpu.VMEM((KCH, C), jnp.int32),
        pltpu.VMEM((KCH, C), jnp.int32),
        pltpu.VMEM((C, D), F32),
        pltpu.VMEM((C, D), F32),
        pltpu.VMEM_SHARED((R, D), F32),
        pltpu.SemaphoreType.DMA,
        pltpu.SemaphoreType.DMA,
    ],
    compiler_params=pltpu.CompilerParams(use_tc_tiling_on_sc=False),
)


def _cnt_body(dstp, cnt_out, dst_v, cnt_v):
    c = lax.axis_index("c")
    s = lax.axis_index("s")
    wid = s * NC + c
    pltpu.sync_copy(dstp.at[wid], dst_v)

    def zc(i, carry):
        cnt_v[pl.ds(i * 16, 16)] = jnp.zeros((16,), F32)
        return carry

    lax.fori_loop(0, R // 16, zc, 0)
    ones16 = jnp.ones((16,), F32)

    def step(j, carry):
        for t in range(C // 16):
            idx16 = dst_v[j, pl.ds(t * 16, 16)]
            plsc.addupdate_scatter(cnt_v, [idx16], ones16)
        return carry

    lax.fori_loop(0, KCH, step, 0)
    for i in range(R // BR):
        pltpu.sync_copy(cnt_v.at[pl.ds(i * BR, BR)], cnt_out.at[i, wid])


_sc_count = pl.kernel(
    _cnt_body,
    out_type=jax.ShapeDtypeStruct((R // BR, NW, BR), F32),
    mesh=plsc.VectorSubcoreMesh(core_axis_name="c", subcore_axis_name="s",
                                num_cores=NC, num_subcores=NS),
    scratch_types=[
        pltpu.VMEM((KCH, C), jnp.int32),
        pltpu.VMEM((R,), F32),
    ],
    compiler_params=pltpu.CompilerParams(needs_layout_passes=False),
)


def _dense_body(acc_ref, cnt_ref, xin_ref, wl_ref, wr_ref, b_ref, out_ref, *, relu):
    acc = acc_ref[0] + acc_ref[1]
    cnt = jnp.sum(cnt_ref[0], axis=0)
    cnt = jnp.maximum(cnt, 1.0).reshape(BR, 1)
    h = (jnp.dot(acc / cnt, wl_ref[...], preferred_element_type=F32)
         + jnp.dot(xin_ref[...], wr_ref[...], preferred_element_type=F32)
         + b_ref[...])
    if relu:
        h = jnp.maximum(h, 0.0)
    out_ref[...] = h


def _dense_layer(acc, cntp, xin, wlT, wrT, b2d, *, relu):
    return pl.pallas_call(
        functools.partial(_dense_body, relu=relu),
        grid=(R // BR,),
        in_specs=[
            pl.BlockSpec((NC, BR, D), lambda i: (0, i, 0)),
            pl.BlockSpec((1, NW, BR), lambda i: (i, 0, 0)),
            pl.BlockSpec((BR, D), lambda i: (i, 0)),
            pl.BlockSpec((D, D), lambda i: (0, 0)),
            pl.BlockSpec((D, D), lambda i: (0, 0)),
            pl.BlockSpec((1, D), lambda i: (0, 0)),
        ],
        out_specs=pl.BlockSpec((BR, D), lambda i: (i, 0)),
        out_shape=jax.ShapeDtypeStruct((R, D), F32),
    )(acc, cntp, xin, wlT, wrT, b2d)


def _pool_head_body(h2_ref, bt_ref, wg1_ref, bg1_ref, wg2_ref, bg2_ref,
                    wo_ref, bo_ref, out_ref):
    run = h2_ref[...]
    b = bt_ref[...]
    d = 1
    while d < R:
        bsh = jnp.concatenate([jnp.full((d, 1), -1, jnp.int32), b[:R - d]], axis=0)
        rsh = jnp.concatenate([run[:d], run[:R - d]], axis=0)
        run = jnp.where(b == bsh, jnp.maximum(run, rsh), run)
        d *= 2
    nxt = jnp.concatenate([b[1:], jnp.full((1, 1), -2, jnp.int32)], axis=0)
    last = (b != nxt)
    onehot = ((b == lax.broadcasted_iota(jnp.int32, (R, G), 1)) & last).astype(F32)
    g = lax.dot_general(onehot, run, (((0,), (0,)), ((), ())),
                        preferred_element_type=F32)
    g = jnp.maximum(jnp.dot(g, wg1_ref[...], preferred_element_type=F32)
                    + bg1_ref[...], 0.0)
    g = jnp.maximum(jnp.dot(g, wg2_ref[...], preferred_element_type=F32)
                    + bg2_ref[...], 0.0)
    out_ref[...] = jnp.dot(g, wo_ref[...], preferred_element_type=F32) + bo_ref[...]


def _pool_head(h2, bt, wg1T, bg1, wg2T, bg2, woT, bo2d):
    return pl.pallas_call(
        _pool_head_body,
        out_shape=jax.ShapeDtypeStruct((G, 1), F32),
    )(h2, bt, wg1T, bg1, wg2T, bg2, woT, bo2d)


def kernel(x, edge_index, batch, W1l, b1l, W1r, W2l, b2l, W2r,
           Wg1, bg1, Wg2, bg2, Wo, bo):
    src = edge_index[0].astype(jnp.int32)
    dst = edge_index[1].astype(jnp.int32)
    srcp = jnp.concatenate(
        [src, jnp.zeros((EPAD - E,), jnp.int32)]).reshape(NW, KCH, C)
    dstp = jnp.concatenate(
        [dst, jnp.full((EPAD - E,), TRASH, jnp.int32)]).reshape(NW, KCH, C)

    xp = jnp.concatenate([x, jnp.zeros((R - N, D), F32)], axis=0)
    zrows = jnp.zeros((ZR, D), F32)

    bt = jnp.concatenate(
        [batch.astype(jnp.int32), jnp.full((R - N,), G, jnp.int32)]
    ).reshape(R, 1)

    cntp = _sc_count(dstp)
    acc1 = _sc_aggregate(xp, srcp, dstp, zrows)
    h1 = _dense_layer(acc1, cntp, xp, W1l.T, W1r.T, b1l.reshape(1, D),
                      relu=True)
    acc2 = _sc_aggregate(h1, srcp, dstp, zrows)
    h2 = _dense_layer(acc2, cntp, h1, W2l.T, W2r.T, b2l.reshape(1, D),
                      relu=False)
    return _pool_head(h2, bt, Wg1.T, bg1.reshape(1, D), Wg2.T,
                      bg2.reshape(1, D), Wo.T, bo.reshape(1, 1))


# speedup vs baseline: 1.0609x; 1.0609x over previous
import functools

import jax
import jax.numpy as jnp
from jax import lax
from jax.experimental import pallas as pl
from jax.experimental.pallas import tpu as pltpu
from jax.experimental.pallas import tpu_sc as plsc

N = 10000
E = 320000
D = 128
G = 128
NC = 2
NS = 16
NW = NC * NS
C = 64
KCH = 160
EPAD = NW * KCH * C
R = 10240
RPT = R // NS
TRASH = 10000
BR = 2560
ZR = 64
F32 = jnp.float32


def _agg_body(table, srcp, dstp, zrows, acc_out,
              src_v, dst_v, rows0, rows1, acc_sh, sem0, sem1):
    rows = (rows0, rows1)
    sems = (sem0, sem1)
    c = lax.axis_index("c")
    s = lax.axis_index("s")
    wid = s * NC + c
    rbase = s * RPT
    for z in range(RPT // ZR):
        pltpu.sync_copy(zrows, acc_sh.at[pl.ds(rbase + z * ZR, ZR)])
    pltpu.sync_copy(srcp.at[wid], src_v)
    pltpu.sync_copy(dstp.at[wid], dst_v)
    pltpu.async_copy(table.at[src_v.at[0]], rows0, sem0)
    pltpu.async_copy(table.at[src_v.at[1]], rows1, sem1)
    plsc.subcore_barrier()

    def step(kk, carry):
        for b in range(2):
            j = 2 * kk + b
            pltpu.make_async_copy(table.at[pl.ds(0, C)], rows[b], sems[b]).wait()
            pltpu.sync_copy(rows[b], acc_sh.at[dst_v.at[j]], add=True)
            jn = jnp.minimum(j + 2, KCH - 1)
            pltpu.async_copy(table.at[src_v.at[jn]], rows[b], sems[b])
        return carry

    lax.fori_loop(0, KCH // 2, step, 0)
    for b in range(2):
        pltpu.make_async_copy(table.at[pl.ds(0, C)], rows[b], sems[b]).wait()
    plsc.subcore_barrier()
    pltpu.sync_copy(acc_sh.at[pl.ds(rbase, RPT)],
                    acc_out.at[c, pl.ds(rbase, RPT)])


_sc_aggregate = pl.kernel(
    _agg_body,
    out_type=jax.ShapeDtypeStruct((NC, R, D), F32),
    mesh=plsc.VectorSubcoreMesh(core_axis_name="c", subcore_axis_name="s",
                                num_cores=NC, num_subcores=NS),
    scratch_types=[
        pltpu.VMEM((KCH, C), jnp.int32),
        pltpu.VMEM((KCH, C), jnp.int32),
        pltpu.VMEM((C, D), F32),
        pltpu.VMEM((C, D), F32),
        pltpu.VMEM_SHARED((R, D), F32),
        pltpu.SemaphoreType.DMA,
        pltpu.SemaphoreType.DMA,
    ],
    compiler_params=pltpu.CompilerParams(use_tc_tiling_on_sc=False),
)


def _cnt_body(dstp, cnt_out, dst_v, cnt_v):
    c = lax.axis_index("c")
    s = lax.axis_index("s")
    wid = s * NC + c
    pltpu.sync_copy(dstp.at[wid], dst_v)

    def zc(i, carry):
        cnt_v[pl.ds(i * 16, 16)] = jnp.zeros((16,), F32)
        return carry

    lax.fori_loop(0, R // 16, zc, 0)
    ones16 = jnp.ones((16,), F32)

    def step(j, carry):
        for t in range(C // 16):
            idx16 = dst_v[j, pl.ds(t * 16, 16)]
            plsc.addupdate_scatter(cnt_v, [idx16], ones16)
        return carry

    lax.fori_loop(0, KCH, step, 0)
    for i in range(R // BR):
        pltpu.sync_copy(cnt_v.at[pl.ds(i * BR, BR)], cnt_out.at[i, wid])


_sc_count = pl.kernel(
    _cnt_body,
    out_type=jax.ShapeDtypeStruct((R // BR, NW, BR), F32),
    mesh=plsc.VectorSubcoreMesh(core_axis_name="c", subcore_axis_name="s",
                                num_cores=NC, num_subcores=NS),
    scratch_types=[
        pltpu.VMEM((KCH, C), jnp.int32),
        pltpu.VMEM((R,), F32),
    ],
    compiler_params=pltpu.CompilerParams(needs_layout_passes=False),
)


def _dense_body(acc_ref, cnt_ref, xin_ref, wl_ref, wr_ref, b_ref, out_ref, *, relu):
    acc = acc_ref[0] + acc_ref[1]
    cnt = jnp.sum(cnt_ref[0], axis=0)
    cnt = jnp.maximum(cnt, 1.0).reshape(BR, 1)
    h = (jnp.dot(acc / cnt, wl_ref[...], preferred_element_type=F32)
         + jnp.dot(xin_ref[...], wr_ref[...], preferred_element_type=F32)
         + b_ref[...])
    if relu:
        h = jnp.maximum(h, 0.0)
    out_ref[...] = h


def _dense_layer(acc, cntp, xin, wlT, wrT, b2d, *, relu):
    return pl.pallas_call(
        functools.partial(_dense_body, relu=relu),
        grid=(R // BR,),
        in_specs=[
            pl.BlockSpec((NC, BR, D), lambda i: (0, i, 0)),
            pl.BlockSpec((1, NW, BR), lambda i: (i, 0, 0)),
            pl.BlockSpec((BR, D), lambda i: (i, 0)),
            pl.BlockSpec((D, D), lambda i: (0, 0)),
            pl.BlockSpec((D, D), lambda i: (0, 0)),
            pl.BlockSpec((1, D), lambda i: (0, 0)),
        ],
        out_specs=pl.BlockSpec((BR, D), lambda i: (i, 0)),
        out_shape=jax.ShapeDtypeStruct((R, D), F32),
    )(acc, cntp, xin, wlT, wrT, b2d)


def _copy_body(x_ref, out_ref):
    out_ref[...] = x_ref[...]


def _tc_copy(xp):
    return pl.pallas_call(
        _copy_body,
        grid=(R // BR,),
        in_specs=[pl.BlockSpec((BR, D), lambda i: (i, 0))],
        out_specs=pl.BlockSpec((BR, D), lambda i: (i, 0)),
        out_shape=jax.ShapeDtypeStruct((R, D), F32),
    )(xp)


def _pool_head_body(h2_ref, bt_ref, wg1_ref, bg1_ref, wg2_ref, bg2_ref,
                    wo_ref, bo_ref, out_ref):
    run = h2_ref[...]
    b = bt_ref[...]
    d = 1
    while d < R:
        bsh = jnp.concatenate([jnp.full((d, 1), -1, jnp.int32), b[:R - d]], axis=0)
        rsh = jnp.concatenate([run[:d], run[:R - d]], axis=0)
        run = jnp.where(b == bsh, jnp.maximum(run, rsh), run)
        d *= 2
    nxt = jnp.concatenate([b[1:], jnp.full((1, 1), -2, jnp.int32)], axis=0)
    last = (b != nxt)
    onehot = ((b == lax.broadcasted_iota(jnp.int32, (R, G), 1)) & last).astype(F32)
    g = lax.dot_general(onehot, run, (((0,), (0,)), ((), ())),
                        preferred_element_type=F32)
    g = jnp.maximum(jnp.dot(g, wg1_ref[...], preferred_element_type=F32)
                    + bg1_ref[...], 0.0)
    g = jnp.maximum(jnp.dot(g, wg2_ref[...], preferred_element_type=F32)
                    + bg2_ref[...], 0.0)
    out_ref[...] = jnp.dot(g, wo_ref[...], preferred_element_type=F32) + bo_ref[...]


def _pool_head(h2, bt, wg1T, bg1, wg2T, bg2, woT, bo2d):
    return pl.pallas_call(
        _pool_head_body,
        out_shape=jax.ShapeDtypeStruct((G, 1), F32),
    )(h2, bt, wg1T, bg1, wg2T, bg2, woT, bo2d)


def kernel(x, edge_index, batch, W1l, b1l, W1r, W2l, b2l, W2r,
           Wg1, bg1, Wg2, bg2, Wo, bo):
    src = edge_index[0].astype(jnp.int32)
    dst = edge_index[1].astype(jnp.int32)
    srcp = jnp.concatenate(
        [src, jnp.zeros((EPAD - E,), jnp.int32)]).reshape(NW, KCH, C)
    dstp = jnp.concatenate(
        [dst, jnp.full((EPAD - E,), TRASH, jnp.int32)]).reshape(NW, KCH, C)

    xp = _tc_copy(jnp.concatenate([x, jnp.zeros((R - N, D), F32)], axis=0))
    zrows = jnp.zeros((ZR, D), F32)

    bt = jnp.concatenate(
        [batch.astype(jnp.int32), jnp.full((R - N,), G, jnp.int32)]
    ).reshape(R, 1)

    acc1 = _sc_aggregate(xp, srcp, dstp, zrows)
    dstp2, _ = lax.optimization_barrier((dstp, acc1))
    cntp = _sc_count(dstp2)
    h1 = _dense_layer(acc1, cntp, xp, W1l.T, W1r.T, b1l.reshape(1, D),
                      relu=True)
    acc2 = _sc_aggregate(h1, srcp, dstp, zrows)
    h2 = _dense_layer(acc2, cntp, h1, W2l.T, W2r.T, b2l.reshape(1, D),
                      relu=False)
    return _pool_head(h2, bt, Wg1.T, bg1.reshape(1, D), Wg2.T,
                      bg2.reshape(1, D), Wo.T, bo.reshape(1, 1))
